# Initial kernel scaffold; baseline (speedup 1.0000x reference)
import jax
import jax.numpy as jnp
from jax.experimental import pallas as pl


def kernel(scalar_feat, coord_feat, vec_feat, edge_index, Wh1, Wu1, ln1_g, ln1_b, lin1_w, lin1_b, gate1_w, gate1_b, Wh2, Wu2, ln2_g, ln2_b, lin2_w, lin2_b, gate2_w, gate2_b, lnorm_g, lnorm_b):
    raise NotImplementedError("write your pallas kernel here")


# speedup vs baseline: 18.4319x; 18.4319x over previous
import functools

import jax
import jax.numpy as jnp
from jax import lax
from jax.experimental import pallas as pl
from jax.experimental.pallas import tpu as pltpu
from jax.experimental.pallas import tpu_sc as plsc

F32 = jnp.float32

ROW = 192
MROW = 176
CROW = 16
CH = 80
NB = 1000
EB = 2000
NC = 2
NS = 16


def _table_body(sf_ref, wgh_ref, p_ref, stats_ref):
    sf = sf_ref[...]
    p_ref[...] = jnp.dot(sf, wgh_ref[...], preferred_element_type=F32)
    stats_ref[:, 0:1] = jnp.sum(sf, axis=1, keepdims=True)
    stats_ref[:, 1:2] = jnp.sum(sf * sf, axis=1, keepdims=True)


def _edge_body(a_ref, b_ref, wh10_ref, wh1v_ref, wu1_ref, wgs_ref, rs_ref,
               c1_ref, g1w_ref, g1b_ref, out_ref):
    a = a_ref[...]
    b = b_ref[...]
    p = a[:, 0:128]
    s1 = a[:, 179:180]
    s2 = a[:, 180:181]

    wh10 = wh10_ref[...]
    wh1v = wh1v_ref[...]
    vh = []
    for c in range(3):
        vc = a[:, 128 + 16 * c:144 + 16 * c]
        dc = a[:, 176 + c:177 + c] - b[:, c:c + 1]
        vh.append(dc * wh10 + jnp.dot(vc, wh1v, preferred_element_type=F32))
    sh_sq = vh[0] * vh[0] + vh[1] * vh[1] + vh[2] * vh[2] + 1e-8
    sh = jnp.sqrt(sh_sq)

    inv_d = 1.0 / 145.0
    m = (s1 + jnp.sum(sh, axis=1, keepdims=True)) * inv_d
    var = (s2 + jnp.sum(sh_sq, axis=1, keepdims=True)) * inv_d - m * m
    inv_std = lax.rsqrt(var + 1e-5)

    t = p + jnp.dot(sh, wgs_ref[...], preferred_element_type=F32)
    y = (t - m * rs_ref[...]) * inv_std + c1_ref[...]
    feats = y * jax.nn.sigmoid(y)

    gates = jnp.dot(feats, g1w_ref[...], preferred_element_type=F32) + g1b_ref[...]
    gv = gates * jax.nn.sigmoid(gates)

    out_ref[:, 0:128] = feats
    wu1 = wu1_ref[...]
    for c in range(3):
        vu = jnp.dot(vh[c], wu1, preferred_element_type=F32)
        out_ref[:, 128 + 16 * c:144 + 16 * c] = gv * vu


def _node_body(s0_ref, vf_ref, acc_ref, lng_ref, lnb_ref, wh2_ref, wu2_ref,
               wg2h_ref, wg2s_ref, rs2_ref, c2_ref, g2w_ref, g2b_ref,
               s_out_ref, v_out_ref):
    acc = acc_ref[...]
    msg = acc[0] + acc[1]
    s = s0_ref[...] + msg[:, 0:128]
    vf = vf_ref[...]
    v = [vf[:, 16 * c:16 * c + 16] + msg[:, 128 + 16 * c:144 + 16 * c]
         for c in range(3)]

    lng = lng_ref[...]
    lnb = lnb_ref[...]
    mu = jnp.mean(s, axis=1, keepdims=True)
    var = jnp.mean(s * s, axis=1, keepdims=True) - mu * mu
    s_ln = (s - mu) * lax.rsqrt(var + 1e-5) * lng + lnb
    vq = v[0] * v[0] + v[1] * v[1] + v[2] * v[2]
    vn = jnp.sqrt(jnp.mean(vq, axis=1, keepdims=True) + 1e-8)
    inv_vn = 1.0 / vn
    vhat = [vc * inv_vn for vc in v]

    wh2 = wh2_ref[...]
    vh2 = [jnp.dot(vc, wh2, preferred_element_type=F32) for vc in vhat]
    sh_sq = vh2[0] * vh2[0] + vh2[1] * vh2[1] + vh2[2] * vh2[2] + 1e-8
    sh = jnp.sqrt(sh_sq)
    inv_d = 1.0 / 145.0
    m2 = (jnp.sum(s_ln, axis=1, keepdims=True)
          + jnp.sum(sh, axis=1, keepdims=True)) * inv_d
    var2 = (jnp.sum(s_ln * s_ln, axis=1, keepdims=True)
            + jnp.sum(sh_sq, axis=1, keepdims=True)) * inv_d - m2 * m2
    inv_std2 = lax.rsqrt(var2 + 1e-5)
    t2 = (jnp.dot(s_ln, wg2h_ref[...], preferred_element_type=F32)
          + jnp.dot(sh, wg2s_ref[...], preferred_element_type=F32))
    y2 = (t2 - m2 * rs2_ref[...]) * inv_std2 + c2_ref[...]
    feats2 = y2 * jax.nn.sigmoid(y2)
    gates2 = jnp.dot(feats2, g2w_ref[...], preferred_element_type=F32) + g2b_ref[...]
    gv2 = gates2 * jax.nn.sigmoid(gates2)

    wu2 = wu2_ref[...]
    s_new = s_ln + feats2
    v_new = []
    for c in range(3):
        vu2 = jnp.dot(vh2[c], wu2, preferred_element_type=F32)
        vres = gv2 * vu2
        v_new.append(vhat[c] + vres[:, 0:16])

    mu3 = jnp.mean(s_new, axis=1, keepdims=True)
    var3 = jnp.mean(s_new * s_new, axis=1, keepdims=True) - mu3 * mu3
    s_out_ref[...] = (s_new - mu3) * lax.rsqrt(var3 + 1e-5) * lng + lnb
    vq3 = v_new[0] * v_new[0] + v_new[1] * v_new[1] + v_new[2] * v_new[2]
    inv_vn3 = 1.0 / jnp.sqrt(jnp.mean(vq3, axis=1, keepdims=True) + 1e-8)
    for c in range(3):
        v_out_ref[:, 16 * c:16 * c + 16] = v_new[c] * inv_vn3


NBUF = 5


def _make_gather(n_edges):
    mesh = plsc.VectorSubcoreMesh(core_axis_name="c", subcore_axis_name="s",
                                  num_cores=NC, num_subcores=NS)
    per_w = n_edges // (NC * NS)
    n_ch = per_w // CH
    assert n_ch % NBUF == 0

    @functools.partial(
        pl.kernel,
        out_type=(jax.ShapeDtypeStruct((n_edges, ROW), F32),
                  jax.ShapeDtypeStruct((n_edges, CROW), F32)),
        mesh=mesh,
        scratch_types=(
            [pltpu.VMEM((CH,), jnp.int32) for _ in range(NBUF)]
            + [pltpu.VMEM((CH,), jnp.int32) for _ in range(NBUF)]
            + [pltpu.VMEM((CH, ROW), F32) for _ in range(NBUF)]
            + [pltpu.VMEM((CH, CROW), F32) for _ in range(NBUF)]
            + [pltpu.SemaphoreType.DMA for _ in range(4 * NBUF)]
        ),
        compiler_params=pltpu.CompilerParams(use_tc_tiling_on_sc=False),
    )
    def gather_k(tbl, ctbl, src, dst, a_out, b_out, *scr):
        idx_s = scr[0:NBUF]
        idx_d = scr[NBUF:2 * NBUF]
        rows = scr[2 * NBUF:3 * NBUF]
        crows = scr[3 * NBUF:4 * NBUF]
        gsa = scr[4 * NBUF:5 * NBUF]
        gsb = scr[5 * NBUF:6 * NBUF]
        wsa = scr[6 * NBUF:7 * NBUF]
        wsb = scr[7 * NBUF:8 * NBUF]
        wid = lax.axis_index("s") * NC + lax.axis_index("c")
        base0 = pl.multiple_of(wid * per_w, 8)

        for b in range(NBUF):
            base = pl.multiple_of(base0 + b * CH, 8)
            pltpu.sync_copy(src.at[pl.ds(base, CH)], idx_s[b])
            pltpu.sync_copy(dst.at[pl.ds(base, CH)], idx_d[b])
            pltpu.async_copy(tbl.at[idx_s[b]], rows[b], gsa[b])
            pltpu.async_copy(ctbl.at[idx_d[b]], crows[b], gsb[b])

        def body(g, carry):
            for b in range(NBUF):
                base = pl.multiple_of(base0 + (g * NBUF + b) * CH, 8)
                pltpu.make_async_copy(tbl.at[idx_s[b]], rows[b], gsa[b]).wait()
                pltpu.make_async_copy(ctbl.at[idx_d[b]], crows[b], gsb[b]).wait()
                pltpu.async_copy(rows[b], a_out.at[pl.ds(base, CH)], wsa[b])
                pltpu.async_copy(crows[b], b_out.at[pl.ds(base, CH)], wsb[b])

                @pl.when(g < (n_ch // NBUF) - 1)
                def _():
                    nbase = pl.multiple_of(base + NBUF * CH, 8)
                    pltpu.sync_copy(src.at[pl.ds(nbase, CH)], idx_s[b])
                    pltpu.sync_copy(dst.at[pl.ds(nbase, CH)], idx_d[b])
                    pltpu.make_async_copy(
                        rows[b], a_out.at[pl.ds(base, CH)], wsa[b]).wait()
                    pltpu.make_async_copy(
                        crows[b], b_out.at[pl.ds(base, CH)], wsb[b]).wait()
                    pltpu.async_copy(tbl.at[idx_s[b]], rows[b], gsa[b])
                    pltpu.async_copy(ctbl.at[idx_d[b]], crows[b], gsb[b])
            return carry

        lax.fori_loop(0, n_ch // NBUF, body, 0)
        last0 = pl.multiple_of(base0 + (n_ch - NBUF) * CH, 8)
        for b in range(NBUF):
            base = pl.multiple_of(last0 + b * CH, 8)
            pltpu.make_async_copy(rows[b], a_out.at[pl.ds(base, CH)], wsa[b]).wait()
            pltpu.make_async_copy(crows[b], b_out.at[pl.ds(base, CH)], wsb[b]).wait()

    return gather_k


CHS = 40
NBS = 2


def _make_scatter(n_nodes, n_edges):
    mesh = plsc.VectorSubcoreMesh(core_axis_name="c", subcore_axis_name="s",
                                  num_cores=NC, num_subcores=NS)
    rows_pt = n_nodes // NS
    per_tile = n_edges // (NC * NS)
    n_ch = per_tile // CHS
    assert n_ch % NBS == 0

    @functools.partial(
        pl.kernel,
        out_type=jax.ShapeDtypeStruct((NC, n_nodes, MROW), F32),
        mesh=mesh,
        scratch_types=(
            [pltpu.VMEM_SHARED((n_nodes, MROW), F32)]
            + [pltpu.VMEM((CHS, MROW), F32) for _ in range(NBS)]
            + [pltpu.VMEM((CHS,), jnp.int32) for _ in range(NBS)]
            + [pltpu.SemaphoreType.DMA for _ in range(2 * NBS)]
        ),
        compiler_params=pltpu.CompilerParams(use_tc_tiling_on_sc=False),
    )
    def scatter_k(m, dstv, zeros, out, accum, *scr):
        mbuf = scr[0:NBS]
        idxb = scr[NBS:2 * NBS]
        msem = scr[2 * NBS:3 * NBS]
        ssem = scr[3 * NBS:4 * NBS]
        cid = lax.axis_index("c")
        sid = lax.axis_index("s")
        pltpu.sync_copy(zeros, accum.at[pl.ds(sid * rows_pt, rows_pt)])
        plsc.subcore_barrier()

        base0 = pl.multiple_of(cid * (n_edges // NC) + sid * per_tile, 8)

        for b in range(NBS):
            base = pl.multiple_of(base0 + b * CHS, 8)
            pltpu.sync_copy(dstv.at[pl.ds(base, CHS)], idxb[b])
            pltpu.async_copy(m.at[pl.ds(base, CHS)], mbuf[b], msem[b])

        def body(g, carry):
            for b in range(NBS):
                base = pl.multiple_of(base0 + (g * NBS + b) * CHS, 8)
                pltpu.make_async_copy(m.at[pl.ds(base, CHS)], mbuf[b],
                                      msem[b]).wait()
                pltpu.async_copy(mbuf[b], accum.at[idxb[b]], ssem[b], add=True)

                @pl.when(g < (n_ch // NBS) - 1)
                def _():
                    nbase = pl.multiple_of(base + NBS * CHS, 8)
                    pltpu.make_async_copy(mbuf[b], accum.at[idxb[b]],
                                          ssem[b]).wait()
                    pltpu.sync_copy(dstv.at[pl.ds(nbase, CHS)], idxb[b])
                    pltpu.async_copy(m.at[pl.ds(nbase, CHS)], mbuf[b], msem[b])
            return carry

        lax.fori_loop(0, n_ch // NBS, body, 0)
        for b in range(NBS):
            pltpu.make_async_copy(mbuf[b], accum.at[idxb[b]], ssem[b]).wait()
        plsc.subcore_barrier()
        pltpu.sync_copy(accum.at[pl.ds(sid * rows_pt, rows_pt)],
                        out.at[cid, pl.ds(sid * rows_pt, rows_pt)])

    return scatter_k


def kernel(scalar_feat, coord_feat, vec_feat, edge_index,
           Wh1, Wu1, ln1_g, ln1_b, lin1_w, lin1_b, gate1_w, gate1_b,
           Wh2, Wu2, ln2_g, ln2_b, lin2_w, lin2_b, gate2_w, gate2_b,
           lnorm_g, lnorm_b):
    n, s_dim = scalar_feat.shape
    e = edge_index.shape[1]
    v_dim = vec_feat.shape[1]
    assert s_dim == 128 and v_dim == 16
    assert n % NB == 0 and e % EB == 0 and e % (NC * NS * CH) == 0
    assert n % NS == 0

    wg1 = lin1_w * ln1_g[None, :]
    wg1h = wg1[:, :s_dim].T
    wg1s = wg1[:, s_dim:].T
    rs1 = jnp.sum(wg1, axis=1)[None, :]
    c1 = (ln1_b @ lin1_w.T + lin1_b)[None, :]
    wh10 = Wh1[0:1, :]
    wh1v = Wh1[1:, :]
    g1w = gate1_w.T
    g1b = gate1_b[None, :]

    wg2 = lin2_w * ln2_g[None, :]
    wg2h = wg2[:, :s_dim].T
    wg2s = wg2[:, s_dim:].T
    rs2 = jnp.sum(wg2, axis=1)[None, :]
    c2 = (ln2_b @ lin2_w.T + lin2_b)[None, :]
    g2w = gate2_w.T
    g2b = gate2_b[None, :]
    lng = lnorm_g[None, :]
    lnb = lnorm_b[None, :]

    vflat = vec_feat.transpose(0, 2, 1).reshape(n, 3 * v_dim)
    src = edge_index[0]
    dst = edge_index[1]

    grid_n = n // NB
    p, stats = pl.pallas_call(
        _table_body,
        grid=(grid_n,),
        in_specs=[
            pl.BlockSpec((NB, s_dim), lambda i: (i, 0)),
            pl.BlockSpec((s_dim, s_dim), lambda i: (0, 0)),
        ],
        out_specs=[
            pl.BlockSpec((NB, s_dim), lambda i: (i, 0)),
            pl.BlockSpec((NB, 16), lambda i: (i, 0)),
        ],
        out_shape=[
            jax.ShapeDtypeStruct((n, s_dim), F32),
            jax.ShapeDtypeStruct((n, 16), F32),
        ],
    )(scalar_feat, wg1h)

    tail = jnp.concatenate(
        [coord_feat, stats[:, 0:1], stats[:, 1:2],
         jnp.zeros((n, ROW - 128 - 48 - 5), F32)], axis=1)
    tbl = jnp.concatenate([p, vflat, tail], axis=1)
    ctbl = jnp.concatenate(
        [coord_feat, jnp.zeros((n, CROW - 3), F32)], axis=1)

    a_edges, b_edges = _make_gather(e)(tbl, ctbl, src, dst)

    grid_e = e // EB
    wspec = lambda shape: pl.BlockSpec(shape, lambda i: tuple(0 for _ in shape))
    m_edges = pl.pallas_call(
        _edge_body,
        grid=(grid_e,),
        in_specs=[
            pl.BlockSpec((EB, ROW), lambda i: (i, 0)),
            pl.BlockSpec((EB, CROW), lambda i: (i, 0)),
            wspec((1, 17)), wspec((16, 17)), wspec((17, 16)),
            wspec((17, 128)), wspec((1, 128)), wspec((1, 128)),
            wspec((128, 16)), wspec((1, 16)),
        ],
        out_specs=pl.BlockSpec((EB, MROW), lambda i: (i, 0)),
        out_shape=jax.ShapeDtypeStruct((e, MROW), F32),
    )(a_edges, b_edges, wh10, wh1v, Wu1, wg1s, rs1, c1, g1w, g1b)

    zeros = jnp.zeros((n // NS, MROW), F32)
    partials = _make_scatter(n, e)(m_edges, dst, zeros)

    s_out, vflat_out = pl.pallas_call(
        _node_body,
        grid=(grid_n,),
        in_specs=[
            pl.BlockSpec((NB, s_dim), lambda i: (i, 0)),
            pl.BlockSpec((NB, 48), lambda i: (i, 0)),
            pl.BlockSpec((NC, NB, MROW), lambda i: (0, i, 0)),
            wspec((1, 128)), wspec((1, 128)),
            wspec((16, 17)), wspec((17, 17)),
            wspec((128, 128)), wspec((17, 128)),
            wspec((1, 128)), wspec((1, 128)),
            wspec((128, 17)), wspec((1, 17)),
        ],
        out_specs=[
            pl.BlockSpec((NB, s_dim), lambda i: (i, 0)),
            pl.BlockSpec((NB, 48), lambda i: (i, 0)),
        ],
        out_shape=[
            jax.ShapeDtypeStruct((n, s_dim), F32),
            jax.ShapeDtypeStruct((n, 48), F32),
        ],
    )(scalar_feat, vflat, partials, lng, lnb, Wh2, Wu2,
      wg2h, wg2s, rs2, c2, g2w, g2b)

    v_out = vflat_out.reshape(n, 3, v_dim).transpose(0, 2, 1)
    return s_out, v_out
